# baseline (speedup 1.0000x reference)
import functools

import jax
import jax.numpy as jnp
from jax import lax
from jax.experimental import pallas as pl
from jax.experimental.pallas import tpu as pltpu
from jax.experimental.pallas import tpu_sc as plsc

D_EMB = 64
L_SEQ = 200
LANES = 16
BLK = 128
NBUF = 4
L_GRPS = 8
B_GRPS = 4


def _sc_embed(x_t, emb_matrix, pe_flat, n_b):
    info = plsc.get_sparse_core_info()
    nc, ns = info.num_cores, info.num_subcores
    nw = nc * ns
    l_per_w = L_SEQ // L_GRPS
    b_per_w = n_b // B_GRPS
    jb_per_l = b_per_w // BLK
    n_blocks = l_per_w * jb_per_l

    mesh = plsc.VectorSubcoreMesh(core_axis_name="c", subcore_axis_name="s")

    @functools.partial(
        pl.kernel,
        out_type=jax.ShapeDtypeStruct((L_SEQ, D_EMB, n_b), jnp.float32),
        mesh=mesh,
        compiler_params=pltpu.CompilerParams(use_tc_tiling_on_sc=False,
                                             needs_layout_passes=False),
        scratch_types=(
            [pltpu.VMEM((BLK,), jnp.int32) for _ in range(NBUF)]
            + [pltpu.VMEM((BLK, D_EMB), jnp.float32) for _ in range(NBUF)]
            + [pltpu.VMEM((D_EMB, BLK), jnp.float32) for _ in range(NBUF)]
            + [pltpu.VMEM((L_SEQ * D_EMB,), jnp.float32)]
            + [pltpu.SemaphoreType.DMA for _ in range(2 * NBUF)]
        ),
    )
    def k(x_hbm, table_hbm, pe_hbm, out_hbm, *scr):
        idxs = scr[:NBUF]
        rows = scr[NBUF:2 * NBUF]
        tbuf = scr[2 * NBUF:3 * NBUF]
        pe_v = scr[3 * NBUF]
        sg = scr[3 * NBUF + 1:3 * NBUF + 1 + NBUF]
        sw = scr[3 * NBUF + 1 + NBUF:]

        wid = lax.axis_index("s") * nc + lax.axis_index("c")
        l0 = (wid // B_GRPS) * l_per_w
        bcol0 = (wid % B_GRPS) * b_per_w
        pltpu.sync_copy(pe_hbm, pe_v)

        def coords(t):
            l = l0 + t // jb_per_l
            bcol = bcol0 + (t % jb_per_l) * BLK
            return l, pl.multiple_of(bcol, BLK)

        def load_idx(t, b):
            l, bcol = coords(t)
            pltpu.sync_copy(x_hbm.at[l, pl.ds(bcol, BLK)], idxs[b])

        def start_gather(b):
            ib = idxs[b]
            rb = rows[b]
            sem = sg[b]

            @plsc.parallel_loop(0, BLK // LANES)
            def gather_body(i):
                idx_vec = ib[pl.ds(i * LANES, LANES)]
                pltpu.async_copy(table_hbm.at[idx_vec],
                                 rb.at[pl.ds(i * LANES, LANES)], sem)

        def wait_gather(b):
            pltpu.make_async_copy(out_hbm.at[0, 0, pl.ds(0, BLK * D_EMB)],
                                  rows[b], sg[b]).wait()

        def fma(t, b):
            rv = rows[b]
            l, _ = coords(t)
            pe = [pe_v[pl.ds(l * D_EMB + c * LANES, LANES)]
                  for c in range(D_EMB // LANES)]

            @plsc.parallel_loop(0, BLK, unroll=4)
            def row_body(j):
                for c in range(D_EMB // LANES):
                    sl = pl.ds(c * LANES, LANES)
                    rv[j, sl] = rv[j, sl] * 0.125 + pe[c]

        def transpose(b):
            rv = rows[b]
            tb = tbuf[b]
            rvecs = [lax.iota(jnp.int32, LANES) + (j0 * LANES)
                     for j0 in range(BLK // LANES)]

            @plsc.parallel_loop(0, D_EMB, unroll=2)
            def col_body(d):
                cvec = jnp.full((LANES,), 0, jnp.int32) + d
                for j0 in range(BLK // LANES):
                    vals = plsc.load_gather(rv, [rvecs[j0], cvec])
                    tb[d, pl.ds(j0 * LANES, LANES)] = vals

        def start_write(t, b):
            l, bcol = coords(t)
            pltpu.async_copy(tbuf[b], out_hbm.at[l, :, pl.ds(bcol, BLK)],
                             sw[b])

        def wait_write(b):
            pltpu.make_async_copy(tbuf[b],
                                  out_hbm.at[0, :, pl.ds(0, BLK)],
                                  sw[b]).wait()

        for t in range(2):
            load_idx(t, t)
            start_gather(t)

        def step_body(s, _):
            for b in range(NBUF):
                t = s * NBUF + b
                wait_gather(b)
                b2 = (b + 2) % NBUF

                @pl.when(t < n_blocks - 2)
                def _():
                    load_idx(t + 2, b2)
                    start_gather(b2)

                fma(t, b)

                @pl.when(t >= NBUF)
                def _():
                    wait_write(b)

                transpose(b)
                start_write(t, b)
            return 0

        lax.fori_loop(0, n_blocks // NBUF, step_body, 0)
        for b in range(NBUF):
            wait_write(b)

    return k(x_t, emb_matrix, pe_flat)


def kernel(x, emb_matrix, pos_enc_max):
    n_b, l = x.shape
    x_t = x.T.astype(jnp.int32)
    pe_flat = (pos_enc_max[:, :l].T * 0.125).astype(jnp.float32).reshape(-1)
    out = _sc_embed(x_t, emb_matrix, pe_flat, n_b)
    return jnp.transpose(out, (2, 0, 1))


# speedup vs baseline: 1.0619x; 1.0619x over previous
import functools

import jax
import jax.numpy as jnp
from jax import lax
from jax.experimental import pallas as pl
from jax.experimental.pallas import tpu as pltpu
from jax.experimental.pallas import tpu_sc as plsc

D_EMB = 64
L_SEQ = 200
LANES = 16
BLK = 128
NBUF = 4
L_GRPS = 8
B_GRPS = 4


def _sc_embed(x_t, emb_matrix, pe_flat, n_b):
    info = plsc.get_sparse_core_info()
    nc, ns = info.num_cores, info.num_subcores
    nw = nc * ns
    l_per_w = L_SEQ // L_GRPS
    b_per_w = n_b // B_GRPS
    jb_per_l = b_per_w // BLK
    n_blocks = l_per_w * jb_per_l

    mesh = plsc.VectorSubcoreMesh(core_axis_name="c", subcore_axis_name="s")

    @functools.partial(
        pl.kernel,
        out_type=jax.ShapeDtypeStruct((L_SEQ, D_EMB, n_b), jnp.float32),
        mesh=mesh,
        compiler_params=pltpu.CompilerParams(use_tc_tiling_on_sc=False,
                                             needs_layout_passes=False),
        scratch_types=(
            [pltpu.VMEM((BLK,), jnp.int32) for _ in range(NBUF)]
            + [pltpu.VMEM((BLK, D_EMB), jnp.float32) for _ in range(NBUF)]
            + [pltpu.VMEM((D_EMB, BLK), jnp.float32) for _ in range(NBUF)]
            + [pltpu.VMEM((L_SEQ * D_EMB,), jnp.float32)]
            + [pltpu.SemaphoreType.DMA for _ in range(3 * NBUF)]
        ),
    )
    def k(x_hbm, table_hbm, pe_hbm, out_hbm, *scr):
        idxs = scr[:NBUF]
        rows = scr[NBUF:2 * NBUF]
        tbuf = scr[2 * NBUF:3 * NBUF]
        pe_v = scr[3 * NBUF]
        sg = scr[3 * NBUF + 1:3 * NBUF + 1 + NBUF]
        sw = scr[3 * NBUF + 1 + NBUF:3 * NBUF + 1 + 2 * NBUF]
        si = scr[3 * NBUF + 1 + 2 * NBUF:]

        wid = lax.axis_index("s") * nc + lax.axis_index("c")
        l0 = (wid // B_GRPS) * l_per_w
        bcol0 = (wid % B_GRPS) * b_per_w
        pltpu.sync_copy(pe_hbm, pe_v)

        def coords(t):
            l = l0 + t // jb_per_l
            bcol = bcol0 + (t % jb_per_l) * BLK
            return l, pl.multiple_of(bcol, BLK)

        def load_idx(t, b):
            l, bcol = coords(t)
            pltpu.async_copy(x_hbm.at[l, pl.ds(bcol, BLK)], idxs[b], si[b])

        def wait_idx(b):
            pltpu.make_async_copy(x_hbm.at[0, pl.ds(0, BLK)], idxs[b],
                                  si[b]).wait()

        def start_gather(b):
            ib = idxs[b]
            rb = rows[b]
            sem = sg[b]

            @plsc.parallel_loop(0, BLK // LANES)
            def gather_body(i):
                idx_vec = ib[pl.ds(i * LANES, LANES)]
                pltpu.async_copy(table_hbm.at[idx_vec],
                                 rb.at[pl.ds(i * LANES, LANES)], sem)

        def wait_gather(b):
            pltpu.make_async_copy(out_hbm.at[0, 0, pl.ds(0, BLK * D_EMB)],
                                  rows[b], sg[b]).wait()

        def fma_transpose(t, b):
            rv = rows[b]
            tb = tbuf[b]
            l, _ = coords(t)
            pe = [pe_v[pl.ds(l * D_EMB + c * LANES, LANES)]
                  for c in range(D_EMB // LANES)]
            cvecs = [lax.iota(jnp.int32, LANES) + (c * LANES)
                     for c in range(D_EMB // LANES)]

            @plsc.parallel_loop(0, BLK, unroll=4)
            def row_body(j):
                jv = jnp.full((LANES,), 0, jnp.int32) + j
                for c in range(D_EMB // LANES):
                    vec = rv[j, pl.ds(c * LANES, LANES)] * 0.125 + pe[c]
                    plsc.store_scatter(tb, [cvecs[c], jv], vec)

        def start_write(t, b):
            l, bcol = coords(t)
            pltpu.async_copy(tbuf[b], out_hbm.at[l, :, pl.ds(bcol, BLK)],
                             sw[b])

        def wait_write(b):
            pltpu.make_async_copy(tbuf[b],
                                  out_hbm.at[0, :, pl.ds(0, BLK)],
                                  sw[b]).wait()

        for t in range(3):
            load_idx(t, t)
        for t in range(2):
            wait_idx(t)
            start_gather(t)

        def step_body(s, _):
            for b in range(NBUF):
                t = s * NBUF + b
                wait_gather(b)
                b2 = (b + 2) % NBUF
                b3 = (b + 3) % NBUF

                @pl.when(t < n_blocks - 3)
                def _():
                    load_idx(t + 3, b3)

                @pl.when(t < n_blocks - 2)
                def _():
                    wait_idx(b2)
                    start_gather(b2)

                @pl.when(t >= NBUF)
                def _():
                    wait_write(b)

                fma_transpose(t, b)
                start_write(t, b)
            return 0

        lax.fori_loop(0, n_blocks // NBUF, step_body, 0)
        for b in range(NBUF):
            wait_write(b)

    return k(x_t, emb_matrix, pe_flat)


def kernel(x, emb_matrix, pos_enc_max):
    n_b, l = x.shape
    x_t = x.T.astype(jnp.int32)
    pe_flat = (pos_enc_max[:, :l].T * 0.125).astype(jnp.float32).reshape(-1)
    out = _sc_embed(x_t, emb_matrix, pe_flat, n_b)
    return jnp.transpose(out, (2, 0, 1))


# speedup vs baseline: 1.3108x; 1.2344x over previous
import functools

import jax
import jax.numpy as jnp
from jax import lax
from jax.experimental import pallas as pl
from jax.experimental.pallas import tpu as pltpu
from jax.experimental.pallas import tpu_sc as plsc

D_EMB = 64
L_SEQ = 200
LANES = 16
CHUNK = 200
NBUF = 4
N_G = CHUNK // LANES


def _sc_embed(x2d, emb_matrix, pe_flat, n_rows):
    info = plsc.get_sparse_core_info()
    nc, ns = info.num_cores, info.num_subcores
    nw = nc * ns
    rows_per_w = n_rows // nw
    n_chunks = rows_per_w // CHUNK

    mesh = plsc.VectorSubcoreMesh(core_axis_name="c", subcore_axis_name="s")

    @functools.partial(
        pl.kernel,
        out_type=jax.ShapeDtypeStruct((n_rows // 2, 2 * D_EMB), jnp.float32),
        mesh=mesh,
        compiler_params=pltpu.CompilerParams(use_tc_tiling_on_sc=False),
        scratch_types=(
            [pltpu.VMEM((CHUNK,), jnp.int32) for _ in range(NBUF)]
            + [pltpu.VMEM((CHUNK, D_EMB), jnp.float32) for _ in range(NBUF)]
            + [pltpu.VMEM((CHUNK // 2, 2 * D_EMB), jnp.float32)
               for _ in range(NBUF)]
            + [pltpu.VMEM((L_SEQ * D_EMB,), jnp.float32)]
            + [pltpu.SemaphoreType.DMA for _ in range(3 * NBUF)]
        ),
    )
    def k(x_hbm, table_hbm, pe_hbm, out_hbm, *scr):
        idxs = scr[:NBUF]
        rows = scr[NBUF:2 * NBUF]
        st = scr[2 * NBUF:3 * NBUF]
        pe_v = scr[3 * NBUF]
        sg = scr[3 * NBUF + 1:3 * NBUF + 1 + NBUF]
        sw = scr[3 * NBUF + 1 + NBUF:3 * NBUF + 1 + 2 * NBUF]
        si = scr[3 * NBUF + 1 + 2 * NBUF:]

        wid = lax.axis_index("s") * nc + lax.axis_index("c")
        seq0 = wid * n_chunks
        stbase = wid * (rows_per_w // 2)
        pltpu.sync_copy(pe_hbm, pe_v)

        def load_idx(g, b):
            pltpu.async_copy(x_hbm.at[seq0 + g], idxs[b], si[b])

        def wait_idx(b):
            pltpu.make_async_copy(x_hbm.at[0], idxs[b], si[b]).wait()

        def start_gather(b):
            ib = idxs[b]
            rb = rows[b]
            sem = sg[b]

            @plsc.parallel_loop(0, N_G)
            def gather_body(i):
                idx_vec = ib[pl.ds(i * LANES, LANES)]
                pltpu.async_copy(table_hbm.at[idx_vec],
                                 rb.at[pl.ds(i * LANES, LANES)], sem)
            tvec = ib[pl.ds(CHUNK - LANES, LANES)]
            pltpu.async_copy(table_hbm.at[tvec],
                             rb.at[pl.ds(CHUNK - LANES, LANES)], sem)

        def wait_gather(b):
            pltpu.make_async_copy(
                table_hbm.at[pl.ds(0, CHUNK)], rows[b], sg[b]).wait()
            pltpu.make_async_copy(
                table_hbm.at[pl.ds(0, 8)],
                rows[b].at[pl.ds(CHUNK - 8, 8)], sg[b]).wait()

        def fma_stage(b):
            rv = rows[b]
            sb = st[b]

            @plsc.parallel_loop(0, CHUNK // 2, unroll=2)
            def row_body(j2):
                for h in range(2):
                    j = j2 * 2 + h
                    for c in range(D_EMB // LANES):
                        sl = pl.ds(c * LANES, LANES)
                        pe = pe_v[pl.ds(j * D_EMB + c * LANES, LANES)]
                        sb[j2, pl.ds(h * D_EMB + c * LANES, LANES)] = (
                            rv[j, sl] * 0.125 + pe)

        def start_write(g, b):
            base = pl.multiple_of(stbase + g * (CHUNK // 2), 4)
            pltpu.async_copy(st[b], out_hbm.at[pl.ds(base, CHUNK // 2)],
                             sw[b])

        def wait_write(b):
            pltpu.make_async_copy(st[b], out_hbm.at[pl.ds(0, CHUNK // 2)],
                                  sw[b]).wait()

        for t in range(3):
            load_idx(t, t)
        for t in range(2):
            wait_idx(t)
            start_gather(t)

        def step_body(s, _):
            for b in range(NBUF):
                g = s * NBUF + b
                wait_gather(b)
                b2 = (b + 2) % NBUF
                b3 = (b + 3) % NBUF

                @pl.when(g < n_chunks - 3)
                def _():
                    load_idx(g + 3, b3)

                @pl.when(g < n_chunks - 2)
                def _():
                    wait_idx(b2)
                    start_gather(b2)

                @pl.when(g >= NBUF)
                def _():
                    wait_write(b)

                fma_stage(b)
                start_write(g, b)
            return 0

        lax.fori_loop(0, n_chunks // NBUF, step_body, 0)
        for b in range(NBUF):
            wait_write(b)

    return k(x2d, emb_matrix, pe_flat)


def kernel(x, emb_matrix, pos_enc_max):
    n_b, l = x.shape
    n_rows = n_b * l
    x2d = x.astype(jnp.int32)
    pe_flat = (pos_enc_max[:, :l].T * 0.125).astype(jnp.float32).reshape(-1)
    out = _sc_embed(x2d, emb_matrix, pe_flat, n_rows)
    return out.reshape(n_b, l, D_EMB)
